# baseline (speedup 1.0000x reference)
import functools

import jax
import jax.numpy as jnp
from jax import lax
from jax.experimental import pallas as pl
from jax.experimental.pallas import tpu as pltpu
from jax.experimental.pallas import tpu_sc as plsc

N = 1_000_000
D = 16
NC = 2
NS = 16
NW = NC * NS
CCH = 1024
G = CCH // 16
NB = 4
NCHUNK = N // CCH
REM0 = NCHUNK * CCH
REM = N - REM0
TT = ((-(-NCHUNK // NW) + NB - 1) // NB) * NB

_mesh = plsc.VectorSubcoreMesh(core_axis_name="c", subcore_axis_name="s")


@functools.partial(
    pl.kernel,
    out_type=(
        jax.ShapeDtypeStruct((NW * 16,), jnp.float32),
        jax.ShapeDtypeStruct((NW * 16,), jnp.int32),
    ),
    mesh=_mesh,
    compiler_params=pltpu.CompilerParams(needs_layout_passes=False),
    scratch_types=(
        pltpu.VMEM((D,), jnp.float32),
        pltpu.VMEM((D, CCH), jnp.float32),
        pltpu.VMEM((D, CCH), jnp.float32),
        pltpu.VMEM((D, CCH), jnp.float32),
        pltpu.VMEM((D, CCH), jnp.float32),
        pltpu.VMEM((D, REM), jnp.float32),
        pltpu.VMEM((16,), jnp.float32),
        pltpu.VMEM((16,), jnp.int32),
        pltpu.SemaphoreType.DMA,
        pltpu.SemaphoreType.DMA,
        pltpu.SemaphoreType.DMA,
        pltpu.SemaphoreType.DMA,
    ),
)
def _nn_scan(query_hbm, keys_hbm, score_out, idx_out,
             qv, buf0, buf1, buf2, buf3, rbuf, sbest, ibest,
             sem0, sem1, sem2, sem3):
    cid = lax.axis_index("c")
    sid = lax.axis_index("s")
    wid = sid * NC + cid

    pltpu.sync_copy(query_hbm, qv)
    q = qv[...]
    qb = [jnp.broadcast_to(q[d], (16,)) for d in range(D)]
    iota = lax.iota(jnp.int32, 16)

    def start(t, buf, sem):
        c = wid + NW * t
        col0 = pl.multiple_of(jnp.where(c < NCHUNK, c * CCH, 0), 128)
        return pltpu.async_copy(keys_hbm.at[:, pl.ds(col0, CCH)], buf, sem)

    def scan_block(buf, base, ngroups, valid, bs, bi):
        def group_body(g, carry):
            bs, bi = carry
            j = g * 32
            a0 = jnp.zeros((16,), jnp.float32)
            b0 = jnp.zeros((16,), jnp.float32)
            a1 = jnp.zeros((16,), jnp.float32)
            b1 = jnp.zeros((16,), jnp.float32)
            for d in range(0, D, 2):
                c0 = buf[d, pl.ds(j, 16)]
                c1 = buf[d, pl.ds(j + 16, 16)]
                c2 = buf[d + 1, pl.ds(j, 16)]
                c3 = buf[d + 1, pl.ds(j + 16, 16)]
                t0 = c0 - qb[d]
                t1 = c1 - qb[d]
                t2 = c2 - qb[d + 1]
                t3 = c3 - qb[d + 1]
                a0 = a0 + t0 * t0
                a1 = a1 + t1 * t1
                b0 = b0 + t2 * t2
                b1 = b1 + t3 * t3
            acc0 = a0 + b0
            acc1 = a1 + b1
            m0 = valid & (acc0 < bs)
            bs = jnp.where(m0, acc0, bs)
            bi = jnp.where(m0, base + j + iota, bi)
            m1 = valid & (acc1 < bs)
            bs = jnp.where(m1, acc1, bs)
            bi = jnp.where(m1, base + j + 16 + iota, bi)
            return bs, bi

        return lax.fori_loop(0, ngroups // 2, group_body, (bs, bi))

    def compute(t, buf, bs, bi):
        c = wid + NW * t
        return scan_block(buf, c * CCH, G, c < NCHUNK, bs, bi)

    bufs = (buf0, buf1, buf2, buf3)
    sems = (sem0, sem1, sem2, sem3)
    for k in range(NB):
        start(k, bufs[k], sems[k])
    bs0 = jnp.full((16,), jnp.inf, jnp.float32)
    bi0 = jnp.zeros((16,), jnp.int32)

    def round_body(tt, carry):
        bs, bi = carry
        t0 = NB * tt
        for k in range(NB):
            pltpu.make_async_copy(
                keys_hbm.at[:, pl.ds(0, CCH)], bufs[k], sems[k]).wait()
            bs, bi = compute(t0 + k, bufs[k], bs, bi)

            @pl.when(t0 + k + NB < TT)
            def _(k=k):
                start(t0 + k + NB, bufs[k], sems[k])

        return bs, bi

    bs, bi = lax.fori_loop(0, TT // NB, round_body, (bs0, bi0))

    @pl.when(wid == NW - 1)
    def _():
        pltpu.sync_copy(keys_hbm.at[:, pl.ds(REM0, REM)], rbuf)
        rbs, rbi = scan_block(rbuf, REM0, REM // 16, True, bs, bi)
        sbest[...] = rbs
        ibest[...] = rbi

    @pl.when(wid != NW - 1)
    def _():
        sbest[...] = bs
        ibest[...] = bi

    pltpu.sync_copy(sbest, score_out.at[pl.ds(wid * 16, 16)])
    pltpu.sync_copy(ibest, idx_out.at[pl.ds(wid * 16, 16)])


@functools.partial(
    pl.kernel,
    out_type=jax.ShapeDtypeStruct((16,), jnp.float32),
    mesh=_mesh,
    compiler_params=pltpu.CompilerParams(needs_layout_passes=False),
    scratch_types=(
        pltpu.VMEM((NW * 16,), jnp.float32),
        pltpu.VMEM((NW * 16,), jnp.int32),
        pltpu.VMEM((D,), jnp.float32),
        pltpu.VMEM((D, 128), jnp.float32),
        pltpu.VMEM((D, N % 128), jnp.float32),
        pltpu.VMEM((16,), jnp.float32),
        pltpu.SemaphoreType.DMA,
    ),
)
def _nn_finish(query_hbm, values_hbm, score_hbm, idx_hbm, out_hbm,
               sbuf, ibuf, qv, vblk, vtail, ob, sem):
    cid = lax.axis_index("c")
    sid = lax.axis_index("s")

    @pl.when((cid == 0) & (sid == 0))
    def _():
        pltpu.sync_copy(score_hbm, sbuf)
        pltpu.sync_copy(idx_hbm, ibuf)
        pltpu.sync_copy(query_hbm, qv)
        bs = sbuf[pl.ds(0, 16)]
        bi = ibuf[pl.ds(0, 16)]
        for r in range(1, NW):
            s = sbuf[pl.ds(r * 16, 16)]
            i = ibuf[pl.ds(r * 16, 16)]
            m = s < bs
            bs = jnp.where(m, s, bs)
            bi = jnp.where(m, i, bi)
        minv = jnp.min(bs)
        rid = jnp.min(jnp.where(bs == minv, bi, jnp.int32(2**31 - 1)))
        base = pl.multiple_of((rid // 128) * 128, 128)
        iota = lax.iota(jnp.int32, 16)
        last0 = (N // 128) * 128

        @pl.when(base < last0)
        def _():
            pltpu.async_copy(
                values_hbm.at[:, pl.ds(base, 128)], vblk, sem).wait()
            sub = jnp.broadcast_to(rid - base, (16,))
            row = plsc.load_gather(vblk, [iota, sub])
            ob[...] = jnp.broadcast_to(jnp.sum(row * qv[...]), (16,))

        @pl.when(base >= last0)
        def _():
            pltpu.async_copy(
                values_hbm.at[:, pl.ds(last0, N % 128)], vtail, sem).wait()
            sub = jnp.broadcast_to(rid - last0, (16,))
            row = plsc.load_gather(vtail, [iota, sub])
            ob[...] = jnp.broadcast_to(jnp.sum(row * qv[...]), (16,))

        pltpu.sync_copy(ob, out_hbm)


def kernel(query, keys, values):
    scores, ids = _nn_scan(query, keys.T)
    out16 = _nn_finish(query, values.T, scores, ids)
    return out16[:1]


# speedup vs baseline: 1.0108x; 1.0108x over previous
import functools

import jax
import jax.numpy as jnp
from jax import lax
from jax.experimental import pallas as pl
from jax.experimental.pallas import tpu as pltpu
from jax.experimental.pallas import tpu_sc as plsc

N = 1_000_000
D = 16
NC = 2
NS = 16
NW = NC * NS
CCH = 1024
G = CCH // 16
NB = 4
NCHUNK = N // CCH
REM0 = NCHUNK * CCH
REM = N - REM0
TT = ((-(-NCHUNK // NW) + NB - 1) // NB) * NB

_mesh = plsc.VectorSubcoreMesh(core_axis_name="c", subcore_axis_name="s")


@functools.partial(
    pl.kernel,
    out_type=(
        jax.ShapeDtypeStruct((NW * 16,), jnp.float32),
        jax.ShapeDtypeStruct((NW * 16,), jnp.int32),
    ),
    mesh=_mesh,
    compiler_params=pltpu.CompilerParams(needs_layout_passes=False),
    scratch_types=(
        pltpu.VMEM((D,), jnp.float32),
        pltpu.VMEM((D, CCH), jnp.float32),
        pltpu.VMEM((D, CCH), jnp.float32),
        pltpu.VMEM((D, CCH), jnp.float32),
        pltpu.VMEM((D, CCH), jnp.float32),
        pltpu.VMEM((D, REM), jnp.float32),
        pltpu.VMEM((16,), jnp.float32),
        pltpu.VMEM((16,), jnp.int32),
        pltpu.SemaphoreType.DMA,
        pltpu.SemaphoreType.DMA,
        pltpu.SemaphoreType.DMA,
        pltpu.SemaphoreType.DMA,
    ),
)
def _nn_scan(query_hbm, keys_hbm, score_out, idx_out,
             qv, buf0, buf1, buf2, buf3, rbuf, sbest, ibest,
             sem0, sem1, sem2, sem3):
    cid = lax.axis_index("c")
    sid = lax.axis_index("s")
    wid = sid * NC + cid

    pltpu.sync_copy(query_hbm, qv)
    q = qv[...]
    qb = [jnp.broadcast_to(q[d], (16,)) for d in range(D)]
    iota = lax.iota(jnp.int32, 16)

    def start(t, buf, sem):
        c = wid + NW * t
        col0 = pl.multiple_of(jnp.where(c < NCHUNK, c * CCH, 0), 128)
        return pltpu.async_copy(keys_hbm.at[:, pl.ds(col0, CCH)], buf, sem)

    def scan_block(buf, base, ngroups, valid, bs, bi):
        def group_body(g, carry):
            bs, bi = carry
            j = g * 32
            acc0 = jnp.zeros((16,), jnp.float32)
            acc1 = jnp.zeros((16,), jnp.float32)
            for d in range(D):
                c0 = buf[d, pl.ds(j, 16)]
                c1 = buf[d, pl.ds(j + 16, 16)]
                t0 = c0 - qb[d]
                t1 = c1 - qb[d]
                acc0 = acc0 + t0 * t0
                acc1 = acc1 + t1 * t1
            m0 = valid & (acc0 < bs)
            bs = jnp.where(m0, acc0, bs)
            bi = jnp.where(m0, base + j + iota, bi)
            m1 = valid & (acc1 < bs)
            bs = jnp.where(m1, acc1, bs)
            bi = jnp.where(m1, base + j + 16 + iota, bi)
            return bs, bi

        return lax.fori_loop(0, ngroups // 2, group_body, (bs, bi))

    def compute(t, buf, bs, bi):
        c = wid + NW * t
        return scan_block(buf, c * CCH, G, c < NCHUNK, bs, bi)

    bufs = (buf0, buf1, buf2, buf3)
    sems = (sem0, sem1, sem2, sem3)
    for k in range(NB):
        start(k, bufs[k], sems[k])
    bs0 = jnp.full((16,), jnp.inf, jnp.float32)
    bi0 = jnp.zeros((16,), jnp.int32)

    def round_body(tt, carry):
        bs, bi = carry
        t0 = NB * tt
        for k in range(NB):
            pltpu.make_async_copy(
                keys_hbm.at[:, pl.ds(0, CCH)], bufs[k], sems[k]).wait()
            bs, bi = compute(t0 + k, bufs[k], bs, bi)

            @pl.when(t0 + k + NB < TT)
            def _(k=k):
                start(t0 + k + NB, bufs[k], sems[k])

        return bs, bi

    bs, bi = lax.fori_loop(0, TT // NB, round_body, (bs0, bi0))

    @pl.when(wid == NW - 1)
    def _():
        pltpu.sync_copy(keys_hbm.at[:, pl.ds(REM0, REM)], rbuf)
        rbs, rbi = scan_block(rbuf, REM0, REM // 16, True, bs, bi)
        sbest[...] = rbs
        ibest[...] = rbi

    @pl.when(wid != NW - 1)
    def _():
        sbest[...] = bs
        ibest[...] = bi

    pltpu.sync_copy(sbest, score_out.at[pl.ds(wid * 16, 16)])
    pltpu.sync_copy(ibest, idx_out.at[pl.ds(wid * 16, 16)])


@functools.partial(
    pl.kernel,
    out_type=jax.ShapeDtypeStruct((16,), jnp.float32),
    mesh=_mesh,
    compiler_params=pltpu.CompilerParams(needs_layout_passes=False),
    scratch_types=(
        pltpu.VMEM((NW * 16,), jnp.float32),
        pltpu.VMEM((NW * 16,), jnp.int32),
        pltpu.VMEM((D,), jnp.float32),
        pltpu.VMEM((D, 128), jnp.float32),
        pltpu.VMEM((D, N % 128), jnp.float32),
        pltpu.VMEM((16,), jnp.float32),
        pltpu.SemaphoreType.DMA,
    ),
)
def _nn_finish(query_hbm, values_hbm, score_hbm, idx_hbm, out_hbm,
               sbuf, ibuf, qv, vblk, vtail, ob, sem):
    cid = lax.axis_index("c")
    sid = lax.axis_index("s")

    @pl.when((cid == 0) & (sid == 0))
    def _():
        pltpu.sync_copy(score_hbm, sbuf)
        pltpu.sync_copy(idx_hbm, ibuf)
        pltpu.sync_copy(query_hbm, qv)
        bs = sbuf[pl.ds(0, 16)]
        bi = ibuf[pl.ds(0, 16)]
        for r in range(1, NW):
            s = sbuf[pl.ds(r * 16, 16)]
            i = ibuf[pl.ds(r * 16, 16)]
            m = s < bs
            bs = jnp.where(m, s, bs)
            bi = jnp.where(m, i, bi)
        minv = jnp.min(bs)
        rid = jnp.min(jnp.where(bs == minv, bi, jnp.int32(2**31 - 1)))
        base = pl.multiple_of((rid // 128) * 128, 128)
        iota = lax.iota(jnp.int32, 16)
        last0 = (N // 128) * 128

        @pl.when(base < last0)
        def _():
            pltpu.async_copy(
                values_hbm.at[:, pl.ds(base, 128)], vblk, sem).wait()
            sub = jnp.broadcast_to(rid - base, (16,))
            row = plsc.load_gather(vblk, [iota, sub])
            ob[...] = jnp.broadcast_to(jnp.sum(row * qv[...]), (16,))

        @pl.when(base >= last0)
        def _():
            pltpu.async_copy(
                values_hbm.at[:, pl.ds(last0, N % 128)], vtail, sem).wait()
            sub = jnp.broadcast_to(rid - last0, (16,))
            row = plsc.load_gather(vtail, [iota, sub])
            ob[...] = jnp.broadcast_to(jnp.sum(row * qv[...]), (16,))

        pltpu.sync_copy(ob, out_hbm)


def kernel(query, keys, values):
    scores, ids = _nn_scan(query, keys.T)
    out16 = _nn_finish(query, values.T, scores, ids)
    return out16[:1]
